# Initial kernel scaffold; baseline (speedup 1.0000x reference)
import jax
import jax.numpy as jnp
from jax.experimental import pallas as pl


def kernel(x, edge_index, rel_ids, emb_rel, W0, b0, W1, b1, W2, b2):
    raise NotImplementedError("write your pallas kernel here")


# speedup vs baseline: 2.4555x; 2.4555x over previous
import functools

import jax
import jax.numpy as jnp
from jax import lax
from jax.experimental import pallas as pl
from jax.experimental.pallas import tpu as pltpu
from jax.experimental.pallas import tpu_sc as plsc

N = 10000
E = 160000
D = 256
NR = 16

NC = 2
NS = 16
NW = NC * NS

CHUNK = 128
EP = 163840
EPW = EP // NS
NCHUNK = EPW // CHUNK
CBUF = 163840
PAD_NODE = 10240

OWN = 320
NOUT = NW * OWN
ACCR = 328
DUMP = 324
FL = 64
CB = 144
SCH = 512
NSCH = EP // SCH
NGRP = SCH // 16

BLK = 400

_SC_PARAMS = pltpu.CompilerParams(needs_layout_passes=False)


def _dot(a, b):
    return lax.dot_general(a, b, (((1,), (0,)), ((), ())),
                           preferred_element_type=jnp.float32,
                           precision=lax.Precision.HIGHEST)


def _sc_counts(nodes2, rel_p):
    mesh = plsc.VectorSubcoreMesh(core_axis_name="c", subcore_axis_name="s")

    @functools.partial(
        pl.kernel,
        out_type=jax.ShapeDtypeStruct((2, CBUF), jnp.float32),
        mesh=mesh,
        compiler_params=_SC_PARAMS,
        scratch_types=[
            pltpu.VMEM((CHUNK,), jnp.int32),
            pltpu.VMEM((CHUNK,), jnp.int32),
            pltpu.VMEM((CHUNK,), jnp.int32),
            pltpu.VMEM((CHUNK,), jnp.float32),
            pltpu.VMEM((2048,), jnp.float32),
            pltpu.VMEM_SHARED((CBUF,), jnp.float32),
            pltpu.SemaphoreType.DMA,
        ],
    )
    def k(nodes_hbm, rel_hbm, out_hbm, nodes_v, rels_v, idx_v, ones_v,
          zeros_v, acc_sh, sem):
        cid = lax.axis_index("c")
        sid = lax.axis_index("s")

        @pl.loop(0, CHUNK, step=16)
        def _(i):
            ones_v[pl.ds(i, 16)] = jnp.full((16,), 1.0, jnp.float32)

        @pl.loop(0, 2048, step=16)
        def _(i):
            zeros_v[pl.ds(i, 16)] = jnp.zeros((16,), jnp.float32)

        @pl.loop(0, CBUF // NS, step=2048)
        def _(j):
            pltpu.sync_copy(zeros_v,
                            acc_sh.at[pl.ds(sid * (CBUF // NS) + j, 2048)])

        plsc.subcore_barrier()

        @pl.loop(0, NCHUNK)
        def _(ci):
            eb = sid * EPW + ci * CHUNK
            pltpu.sync_copy(nodes_hbm.at[cid, pl.ds(eb, CHUNK)], nodes_v)
            pltpu.sync_copy(rel_hbm.at[pl.ds(eb, CHUNK)], rels_v)

            @pl.loop(0, CHUNK, step=16)
            def _(i):
                nv = nodes_v[pl.ds(i, 16)]
                rv = rels_v[pl.ds(i, 16)]
                idx_v[pl.ds(i, 16)] = nv * NR + rv

            pltpu.sync_copy(ones_v, acc_sh.at[idx_v], add=True)

        plsc.subcore_barrier()

        @pl.loop(0, CBUF // NS, step=2048)
        def _(j):
            off = sid * (CBUF // NS) + j
            pltpu.sync_copy(acc_sh.at[pl.ds(off, 2048)],
                            out_hbm.at[cid, pl.ds(off, 2048)])

    return k(nodes2, rel_p)


def _sc_agg(h_aug, src_g, dst_p):
    mesh = plsc.VectorSubcoreMesh(core_axis_name="c", subcore_axis_name="s")

    @functools.partial(
        pl.kernel,
        out_type=jax.ShapeDtypeStruct((NOUT, D), jnp.float32),
        mesh=mesh,
        compiler_params=_SC_PARAMS,
        scratch_types=[
            pltpu.VMEM((SCH,), jnp.int32),
            pltpu.VMEM((SCH,), jnp.int32),
            pltpu.VMEM((CB,), jnp.int32),
            pltpu.VMEM((CB,), jnp.int32),
            pltpu.VMEM((FL,), jnp.int32),
            pltpu.VMEM((FL,), jnp.int32),
            pltpu.VMEM((FL, D), jnp.float32),
            pltpu.VMEM((ACCR, D), jnp.float32),
            pltpu.SemaphoreType.DMA,
        ],
    )
    def k(h_hbm, src_hbm, dst_hbm, out_hbm, sch_v, dch_v, psrc_v, pdl_v,
          fsrc_v, fdl_v, grows_v, acc_v, sem):
        cid = lax.axis_index("c")
        sid = lax.axis_index("s")
        w = sid * NC + cid
        rbase = w * OWN
        col16 = lax.iota(jnp.int32, 16)

        @pl.loop(0, ACCR)
        def _(r):
            for j in range(0, D, 16):
                acc_v[r, pl.ds(j, 16)] = jnp.zeros((16,), jnp.float32)

        def flush(off):
            for j in range(0, FL, 16):
                fsrc_v[pl.ds(j, 16)] = psrc_v[pl.ds(j, 16)]
                fdl_v[pl.ds(j, 16)] = pdl_v[pl.ds(j, 16)]
            pltpu.async_copy(h_hbm.at[fsrc_v], grows_v, sem).wait()

            def acc_body(e, carry):
                eb = jnp.full((16,), e, jnp.int32)
                row16 = plsc.load_gather(fdl_v, [eb])
                for kk in range(0, D, 16):
                    vals = grows_v[e, pl.ds(kk, 16)]
                    plsc.addupdate_scatter(acc_v, [row16, col16 + kk], vals)
                return carry

            lax.fori_loop(0, FL, acc_body, 0)
            psrc_v[pl.ds(0, 16)] = psrc_v[pl.ds(FL, 16)]
            pdl_v[pl.ds(0, 16)] = pdl_v[pl.ds(FL, 16)]
            return off - FL

        def group(g, off, base):
            d16 = dch_v[pl.ds(g * 16, 16)]
            s16 = sch_v[pl.ds(g * 16, 16)]
            dl = d16 - rbase
            m = (dl >= 0) & (dl < OWN)
            m32 = m.astype(jnp.int32)
            pos = off + plsc.cumsum(m32) - 1
            plsc.store_scatter(psrc_v, [pos], s16, mask=m)
            plsc.store_scatter(pdl_v, [pos], dl, mask=m)
            off = off + jnp.sum(m32)
            return lax.cond(off >= FL, flush, lambda o: o, off)

        def chunk(ci, off):
            eb = ci * SCH
            pltpu.sync_copy(src_hbm.at[pl.ds(eb, SCH)], sch_v)
            pltpu.sync_copy(dst_hbm.at[pl.ds(eb, SCH)], dch_v)

            def grp_body(g, off):
                return group(g, off, eb)

            return lax.fori_loop(0, NGRP, grp_body, off)

        off = lax.fori_loop(0, NSCH, chunk, jnp.int32(0))

        def pad_body(t, off_c):
            psrc_v[pl.ds(off_c + t * 16, 16)] = jnp.zeros((16,), jnp.int32)
            pdl_v[pl.ds(off_c + t * 16, 16)] = jnp.full((16,), DUMP, jnp.int32)
            return off_c

        lax.fori_loop(0, 5, pad_body, off)
        flush(off)

        plsc.subcore_barrier()
        pltpu.sync_copy(acc_v.at[pl.ds(0, OWN)],
                        out_hbm.at[pl.ds(rbase, OWN)])

    return k(h_aug, src_g, dst_p)


def _tc_input(x, W0, b0, c_src, emb_rel):
    def body(x_ref, w_ref, b_ref, c_ref, e_ref, h_ref, ha_ref):
        hb = jnp.tanh(_dot(x_ref[...], w_ref[...]) + b_ref[...])
        h_ref[...] = hb
        ha_ref[...] = hb + _dot(c_ref[...], e_ref[...])

    return pl.pallas_call(
        body,
        grid=(N // BLK,),
        in_specs=[
            pl.BlockSpec((BLK, D), lambda i: (i, 0)),
            pl.BlockSpec((D, D), lambda i: (0, 0)),
            pl.BlockSpec((1, D), lambda i: (0, 0)),
            pl.BlockSpec((BLK, NR), lambda i: (i, 0)),
            pl.BlockSpec((NR, D), lambda i: (0, 0)),
        ],
        out_specs=[
            pl.BlockSpec((BLK, D), lambda i: (i, 0)),
            pl.BlockSpec((BLK, D), lambda i: (i, 0)),
        ],
        out_shape=[
            jax.ShapeDtypeStruct((N, D), jnp.float32),
            jax.ShapeDtypeStruct((N, D), jnp.float32),
        ],
    )(x, W0, b0.reshape(1, D), c_src, emb_rel)


def _tc_output(agg, c_dst, emb_rel, h, W1, b1, W2, b2):
    def body(a_ref, c_ref, e_ref, h_ref, w1_ref, b1_ref, w2_ref, b2_ref,
             o_ref):
        cb = c_ref[...]
        deg = jnp.maximum(jnp.sum(cb, axis=1, keepdims=True), 1.0)
        t = a_ref[...] / deg
        u = jnp.tanh(_dot(t, w1_ref[...]) + b1_ref[...] + h_ref[...]
                     + _dot(cb, e_ref[...]))
        o_ref[...] = _dot(u, w2_ref[...]) + b2_ref[...]

    return pl.pallas_call(
        body,
        grid=(N // BLK,),
        in_specs=[
            pl.BlockSpec((BLK, D), lambda i: (i, 0)),
            pl.BlockSpec((BLK, NR), lambda i: (i, 0)),
            pl.BlockSpec((NR, D), lambda i: (0, 0)),
            pl.BlockSpec((BLK, D), lambda i: (i, 0)),
            pl.BlockSpec((D, D), lambda i: (0, 0)),
            pl.BlockSpec((1, D), lambda i: (0, 0)),
            pl.BlockSpec((D, D), lambda i: (0, 0)),
            pl.BlockSpec((1, D), lambda i: (0, 0)),
        ],
        out_specs=pl.BlockSpec((BLK, D), lambda i: (i, 0)),
        out_shape=jax.ShapeDtypeStruct((N, D), jnp.float32),
    )(agg, c_dst, emb_rel, h, W1, b1.reshape(1, D), W2, b2.reshape(1, D))


def kernel(x, edge_index, rel_ids, emb_rel, W0, b0, W1, b1, W2, b2):
    src = edge_index[0].astype(jnp.int32)
    dst = edge_index[1].astype(jnp.int32)
    rel = rel_ids.astype(jnp.int32)

    pad = EP - E
    src_cnt = jnp.concatenate([src, jnp.full((pad,), N, jnp.int32)])
    dst_cnt = jnp.concatenate([dst, jnp.full((pad,), N, jnp.int32)])
    dst_p = jnp.concatenate([dst, jnp.full((pad,), PAD_NODE, jnp.int32)])
    src_g = jnp.concatenate([src, jnp.zeros((pad,), jnp.int32)])
    rel_p = jnp.concatenate([rel, jnp.zeros((pad,), jnp.int32)])
    nodes2 = jnp.stack([src_cnt, dst_cnt])

    counts = _sc_counts(nodes2, rel_p)
    c_src = counts[0, : N * NR].reshape(N, NR)
    c_dst = counts[1, : N * NR].reshape(N, NR)

    h, h_aug = _tc_input(x, W0, b0, c_src, emb_rel)
    agg = _sc_agg(h_aug, src_g, dst_p)[:N]
    return _tc_output(agg, c_dst, emb_rel, h, W1, b1, W2, b2)
